# Initial kernel scaffold; baseline (speedup 1.0000x reference)
import jax
import jax.numpy as jnp
from jax.experimental import pallas as pl


def kernel(x, edge_index, batch, W1_rel, W1_root, b1, W2_rel, W2_root, b2, W3_rel, W3_root, b3, lin_W, lin_b):
    raise NotImplementedError("write your pallas kernel here")


# speedup vs baseline: 7.7798x; 7.7798x over previous
import functools

import jax
import jax.numpy as jnp
from jax import lax
from jax.experimental import pallas as pl
from jax.experimental.pallas import tpu as pltpu
from jax.experimental.pallas import tpu_sc as plsc

N = 10000
E = 320000
D = 128
H = 128
G = 64
C = 10

NC = 2
NS = 16
NW = NC * NS
CH = 125
EPW = E // NW
NCH = EPW // CH
RPW = N // NS
ZR = 25
WBW = 10
WBR = N // WBW

BN = 2000
NB = N // BN


def _segsum_body(src_hbm, dst_hbm, y_hbm, out_hbm, src_v, dst_v, rows_v,
                 zbuf, acc, sem):
    c = lax.axis_index("c")
    s = lax.axis_index("s")
    wid = s * NC + c

    def _zb(t, carry):
        i = t // (D // 16)
        j = t % (D // 16)
        zbuf[i, pl.ds(j * 16, 16)] = jnp.zeros((16,), jnp.float32)
        return carry
    lax.fori_loop(0, ZR * (D // 16), _zb, 0)
    for r in range(RPW // ZR):
        pltpu.sync_copy(zbuf, acc.at[pl.ds(s * RPW + r * ZR, ZR)])
    plsc.subcore_barrier()

    pltpu.sync_copy(src_hbm.at[pl.ds(wid * NCH, NCH)], src_v)
    pltpu.sync_copy(dst_hbm.at[pl.ds(wid * NCH, NCH)], dst_v)

    def _edge(i, carry):
        pltpu.async_copy(y_hbm.at[src_v.at[i]], rows_v, sem).wait()
        pltpu.sync_copy(rows_v, acc.at[dst_v.at[i]], add=True)
        return carry
    lax.fori_loop(0, NCH, _edge, 0)
    plsc.subcore_barrier()

    @pl.when(s < WBW)
    def _():
        pltpu.sync_copy(acc.at[pl.ds(s * WBR, WBR)],
                        out_hbm.at[c, pl.ds(s * WBR, WBR)])


_segsum = pl.kernel(
    _segsum_body,
    out_type=jax.ShapeDtypeStruct((NC, N, H), jnp.float32),
    mesh=plsc.VectorSubcoreMesh(core_axis_name="c", subcore_axis_name="s"),
    scratch_types=[
        pltpu.VMEM((NCH, CH), jnp.int32),
        pltpu.VMEM((NCH, CH), jnp.int32),
        pltpu.VMEM((CH, H), jnp.float32),
        pltpu.VMEM((ZR, D), jnp.float32),
        pltpu.VMEM_SHARED((N, H), jnp.float32),
        pltpu.SemaphoreType.DMA,
    ],
)


def _mm_body(x_ref, w_ref, o_ref):
    o_ref[...] = jnp.dot(x_ref[...], w_ref[...],
                         preferred_element_type=jnp.float32)


_mm = pl.pallas_call(
    _mm_body,
    grid=(NB,),
    in_specs=[
        pl.BlockSpec((BN, D), lambda i: (i, 0)),
        pl.BlockSpec((D, H), lambda i: (0, 0)),
    ],
    out_specs=pl.BlockSpec((BN, H), lambda i: (i, 0)),
    out_shape=jax.ShapeDtypeStruct((N, H), jnp.float32),
)


def _fuse_body(agg_ref, h_ref, wroot_ref, b_ref, wrel_ref, ho_ref, yo_ref):
    a = agg_ref[0] + agg_ref[1]
    t = a + jnp.dot(h_ref[...], wroot_ref[...],
                    preferred_element_type=jnp.float32) + b_ref[...]
    h = jnp.maximum(t, 0.0)
    ho_ref[...] = h
    yo_ref[...] = jnp.dot(h, wrel_ref[...], preferred_element_type=jnp.float32)


_fuse = pl.pallas_call(
    _fuse_body,
    grid=(NB,),
    in_specs=[
        pl.BlockSpec((NC, BN, H), lambda i: (0, i, 0)),
        pl.BlockSpec((BN, H), lambda i: (i, 0)),
        pl.BlockSpec((H, H), lambda i: (0, 0)),
        pl.BlockSpec((1, H), lambda i: (0, 0)),
        pl.BlockSpec((H, H), lambda i: (0, 0)),
    ],
    out_specs=[
        pl.BlockSpec((BN, H), lambda i: (i, 0)),
        pl.BlockSpec((BN, H), lambda i: (i, 0)),
    ],
    out_shape=[
        jax.ShapeDtypeStruct((N, H), jnp.float32),
        jax.ShapeDtypeStruct((N, H), jnp.float32),
    ],
)


def _final_body(agg_ref, h_ref, wroot_ref, b_ref, batch_ref, linw_ref,
                linb_ref, o_ref, sums, cnts):
    i = pl.program_id(0)

    @pl.when(i == 0)
    def _():
        sums[...] = jnp.zeros_like(sums)
        cnts[...] = jnp.zeros_like(cnts)

    a = agg_ref[0] + agg_ref[1]
    h3 = a + jnp.dot(h_ref[...], wroot_ref[...],
                     preferred_element_type=jnp.float32) + b_ref[...]
    bb = batch_ref[0, 0, :]
    onehot = (bb[:, None] == lax.broadcasted_iota(jnp.int32, (1, G), 1)
              ).astype(jnp.float32)
    dn = (((0,), (0,)), ((), ()))
    sums[...] += lax.dot_general(onehot, h3, dn,
                                 preferred_element_type=jnp.float32)
    cnts[...] += lax.dot_general(onehot, jnp.ones_like(h3), dn,
                                 preferred_element_type=jnp.float32)

    @pl.when(i == NB - 1)
    def _():
        pooled = sums[...] / jnp.maximum(cnts[...], 1.0)
        o_ref[...] = jnp.dot(pooled, linw_ref[...],
                             preferred_element_type=jnp.float32) + linb_ref[...]


_final = pl.pallas_call(
    _final_body,
    grid=(NB,),
    in_specs=[
        pl.BlockSpec((NC, BN, H), lambda i: (0, i, 0)),
        pl.BlockSpec((BN, H), lambda i: (i, 0)),
        pl.BlockSpec((H, H), lambda i: (0, 0)),
        pl.BlockSpec((1, H), lambda i: (0, 0)),
        pl.BlockSpec((1, 1, BN), lambda i: (i, 0, 0)),
        pl.BlockSpec((H, C), lambda i: (0, 0)),
        pl.BlockSpec((1, C), lambda i: (0, 0)),
    ],
    out_specs=pl.BlockSpec((G, C), lambda i: (0, 0)),
    out_shape=jax.ShapeDtypeStruct((G, C), jnp.float32),
    scratch_shapes=[
        pltpu.VMEM((G, H), jnp.float32),
        pltpu.VMEM((G, H), jnp.float32),
    ],
)


def kernel(x, edge_index, batch, W1_rel, W1_root, b1, W2_rel, W2_root, b2,
           W3_rel, W3_root, b3, lin_W, lin_b):
    src = edge_index[0].reshape(NW * NCH, CH)
    dst = edge_index[1].reshape(NW * NCH, CH)
    batch_r = batch.reshape(NB, 1, BN)

    y1 = _mm(x, W1_rel)
    agg1 = _segsum(src, dst, y1)
    h1, y2 = _fuse(agg1, x, W1_root, b1.reshape(1, H), W2_rel)
    agg2 = _segsum(src, dst, y2)
    h2, y3 = _fuse(agg2, h1, W2_root, b2.reshape(1, H), W3_rel)
    agg3 = _segsum(src, dst, y3)
    out = _final(agg3, h2, W3_root, b3.reshape(1, H), batch_r, lin_W,
                 lin_b.reshape(1, C))
    return out


# speedup vs baseline: 11.6356x; 1.4956x over previous
import functools

import jax
import jax.numpy as jnp
from jax import lax
from jax.experimental import pallas as pl
from jax.experimental.pallas import tpu as pltpu
from jax.experimental.pallas import tpu_sc as plsc

N = 10000
E = 320000
D = 128
H = 128
G = 64
C = 10

NC = 2
NS = 16
NW = NC * NS
CH = 125
EPW = E // NW
NCH = EPW // CH
WIN = 40
RPW = N // NS
ZR = 25
WBW = 10
WBR = N // WBW

BN = 2000
NB = N // BN


def _segsum_body(src_hbm, dst_hbm, y_hbm, out_hbm, src_v, dst_v, rows_a,
                 rows_b, zbuf, acc, sem_a, sem_b):
    c = lax.axis_index("c")
    s = lax.axis_index("s")
    wid = s * NC + c

    def _zb(t, carry):
        i = t // (D // 16)
        j = t % (D // 16)
        zbuf[i, pl.ds(j * 16, 16)] = jnp.zeros((16,), jnp.float32)
        return carry
    lax.fori_loop(0, ZR * (D // 16), _zb, 0)
    for r in range(RPW // ZR):
        pltpu.sync_copy(zbuf, acc.at[pl.ds(s * RPW + r * ZR, ZR)])
    plsc.subcore_barrier()

    for w in range(NCH // WIN):
        pltpu.sync_copy(src_hbm.at[pl.ds(wid * NCH + w * WIN, WIN)], src_v)
        pltpu.sync_copy(dst_hbm.at[pl.ds(wid * NCH + w * WIN, WIN)], dst_v)
        pltpu.async_copy(y_hbm.at[src_v.at[0]], rows_a, sem_a)

        def _edge(k, carry):
            i = 2 * k
            cpb = pltpu.async_copy(y_hbm.at[src_v.at[i + 1]], rows_b, sem_b)
            pltpu.make_async_copy(y_hbm.at[src_v.at[i]], rows_a, sem_a).wait()
            pltpu.sync_copy(rows_a, acc.at[dst_v.at[i]], add=True)

            @pl.when(i + 2 < WIN)
            def _():
                pltpu.async_copy(y_hbm.at[src_v.at[i + 2]], rows_a, sem_a)

            cpb.wait()
            pltpu.sync_copy(rows_b, acc.at[dst_v.at[i + 1]], add=True)
            return carry
        lax.fori_loop(0, WIN // 2, _edge, 0)
    plsc.subcore_barrier()

    @pl.when(s < WBW)
    def _():
        pltpu.sync_copy(acc.at[pl.ds(s * WBR, WBR)],
                        out_hbm.at[c, pl.ds(s * WBR, WBR)])


_segsum = pl.kernel(
    _segsum_body,
    out_type=jax.ShapeDtypeStruct((NC, N, H), jnp.float32),
    mesh=plsc.VectorSubcoreMesh(core_axis_name="c", subcore_axis_name="s"),
    scratch_types=[
        pltpu.VMEM((WIN, CH), jnp.int32),
        pltpu.VMEM((WIN, CH), jnp.int32),
        pltpu.VMEM((CH, H), jnp.float32),
        pltpu.VMEM((CH, H), jnp.float32),
        pltpu.VMEM((ZR, D), jnp.float32),
        pltpu.VMEM_SHARED((N, H), jnp.float32),
        pltpu.SemaphoreType.DMA,
        pltpu.SemaphoreType.DMA,
    ],
)


def _mm_body(x_ref, w_ref, o_ref):
    o_ref[...] = jnp.dot(x_ref[...], w_ref[...],
                         preferred_element_type=jnp.float32)


_mm = pl.pallas_call(
    _mm_body,
    grid=(NB,),
    in_specs=[
        pl.BlockSpec((BN, D), lambda i: (i, 0)),
        pl.BlockSpec((D, H), lambda i: (0, 0)),
    ],
    out_specs=pl.BlockSpec((BN, H), lambda i: (i, 0)),
    out_shape=jax.ShapeDtypeStruct((N, H), jnp.float32),
)


def _fuse_body(agg_ref, h_ref, wroot_ref, b_ref, wrel_ref, ho_ref, yo_ref):
    a = agg_ref[0] + agg_ref[1]
    t = a + jnp.dot(h_ref[...], wroot_ref[...],
                    preferred_element_type=jnp.float32) + b_ref[...]
    h = jnp.maximum(t, 0.0)
    ho_ref[...] = h
    yo_ref[...] = jnp.dot(h, wrel_ref[...], preferred_element_type=jnp.float32)


_fuse = pl.pallas_call(
    _fuse_body,
    grid=(NB,),
    in_specs=[
        pl.BlockSpec((NC, BN, H), lambda i: (0, i, 0)),
        pl.BlockSpec((BN, H), lambda i: (i, 0)),
        pl.BlockSpec((H, H), lambda i: (0, 0)),
        pl.BlockSpec((1, H), lambda i: (0, 0)),
        pl.BlockSpec((H, H), lambda i: (0, 0)),
    ],
    out_specs=[
        pl.BlockSpec((BN, H), lambda i: (i, 0)),
        pl.BlockSpec((BN, H), lambda i: (i, 0)),
    ],
    out_shape=[
        jax.ShapeDtypeStruct((N, H), jnp.float32),
        jax.ShapeDtypeStruct((N, H), jnp.float32),
    ],
)


def _final_body(agg_ref, h_ref, wroot_ref, b_ref, batch_ref, linw_ref,
                linb_ref, o_ref, sums, cnts):
    i = pl.program_id(0)

    @pl.when(i == 0)
    def _():
        sums[...] = jnp.zeros_like(sums)
        cnts[...] = jnp.zeros_like(cnts)

    a = agg_ref[0] + agg_ref[1]
    h3 = a + jnp.dot(h_ref[...], wroot_ref[...],
                     preferred_element_type=jnp.float32) + b_ref[...]
    bb = batch_ref[0, 0, :]
    onehot = (bb[:, None] == lax.broadcasted_iota(jnp.int32, (1, G), 1)
              ).astype(jnp.float32)
    dn = (((0,), (0,)), ((), ()))
    sums[...] += lax.dot_general(onehot, h3, dn,
                                 preferred_element_type=jnp.float32)
    cnts[...] += lax.dot_general(onehot, jnp.ones_like(h3), dn,
                                 preferred_element_type=jnp.float32)

    @pl.when(i == NB - 1)
    def _():
        pooled = sums[...] / jnp.maximum(cnts[...], 1.0)
        o_ref[...] = jnp.dot(pooled, linw_ref[...],
                             preferred_element_type=jnp.float32) + linb_ref[...]


_final = pl.pallas_call(
    _final_body,
    grid=(NB,),
    in_specs=[
        pl.BlockSpec((NC, BN, H), lambda i: (0, i, 0)),
        pl.BlockSpec((BN, H), lambda i: (i, 0)),
        pl.BlockSpec((H, H), lambda i: (0, 0)),
        pl.BlockSpec((1, H), lambda i: (0, 0)),
        pl.BlockSpec((1, 1, BN), lambda i: (i, 0, 0)),
        pl.BlockSpec((H, C), lambda i: (0, 0)),
        pl.BlockSpec((1, C), lambda i: (0, 0)),
    ],
    out_specs=pl.BlockSpec((G, C), lambda i: (0, 0)),
    out_shape=jax.ShapeDtypeStruct((G, C), jnp.float32),
    scratch_shapes=[
        pltpu.VMEM((G, H), jnp.float32),
        pltpu.VMEM((G, H), jnp.float32),
    ],
)


def kernel(x, edge_index, batch, W1_rel, W1_root, b1, W2_rel, W2_root, b2,
           W3_rel, W3_root, b3, lin_W, lin_b):
    src = edge_index[0].reshape(NW * NCH, CH)
    dst = edge_index[1].reshape(NW * NCH, CH)
    batch_r = batch.reshape(NB, 1, BN)

    y1 = _mm(x, W1_rel)
    agg1 = _segsum(src, dst, y1)
    h1, y2 = _fuse(agg1, x, W1_root, b1.reshape(1, H), W2_rel)
    agg2 = _segsum(src, dst, y2)
    h2, y3 = _fuse(agg2, h1, W2_root, b2.reshape(1, H), W3_rel)
    agg3 = _segsum(src, dst, y3)
    out = _final(agg3, h2, W3_root, b3.reshape(1, H), batch_r, lin_W,
                 lin_b.reshape(1, C))
    return out
